# Initial kernel scaffold; baseline (speedup 1.0000x reference)
import jax
import jax.numpy as jnp
from jax.experimental import pallas as pl


def kernel(x, relation_embedding, W_src, W_dst, rel_trans_w, res_w, res_b, residual_weight, rel_prop_w, rel_prop_b, edge_index):
    raise NotImplementedError("write your pallas kernel here")


# speedup vs baseline: 26.2131x; 26.2131x over previous
import jax
import jax.numpy as jnp
from jax import lax
from jax.experimental import pallas as pl
from jax.experimental.pallas import tpu as pltpu
from jax.experimental.pallas import tpu_sc as plsc

N = 50000
NP = 50048
E = 800000
DIM = 64
H = 8
HD = 8
NEG = 0.2
NS = 16
CH = 2000
NCH = E // CH
GP16 = CH // 16
RB = 1000
GRID = N // RB


def _proj_body(x_ref, wsrc_ref, esw_ref, edw_ref, fs_ref, es_ref, ed_ref):
    xb = x_ref[...]
    fs_ref[...] = jnp.dot(xb, wsrc_ref[...], preferred_element_type=jnp.float32)
    es_ref[...] = jnp.dot(xb, esw_ref[...], preferred_element_type=jnp.float32)
    ed_ref[...] = jnp.dot(xb, edw_ref[...], preferred_element_type=jnp.float32)


def _run_proj(x, W_src, EsrcW, EdstW):
    return pl.pallas_call(
        _proj_body,
        grid=(GRID,),
        in_specs=[
            pl.BlockSpec((RB, DIM), lambda i: (i, 0)),
            pl.BlockSpec((DIM, DIM), lambda i: (0, 0)),
            pl.BlockSpec((DIM, H), lambda i: (0, 0)),
            pl.BlockSpec((DIM, H), lambda i: (0, 0)),
        ],
        out_specs=[
            pl.BlockSpec((RB, DIM), lambda i: (i, 0)),
            pl.BlockSpec((RB, H), lambda i: (i, 0)),
            pl.BlockSpec((RB, H), lambda i: (i, 0)),
        ],
        out_shape=[
            jax.ShapeDtypeStruct((N, DIM), jnp.float32),
            jax.ShapeDtypeStruct((N, H), jnp.float32),
            jax.ShapeDtypeStruct((N, H), jnp.float32),
        ],
    )(x, W_src, EsrcW, EdstW)


def _sc_edge_body(esT, edT, featT, srcidx, dstidx,
                  exout, accout,
                  colbuf, acol, sbuf, dbuf, exb):
    c = lax.axis_index("c")
    s = lax.axis_index("s")
    tid = c * NS + s

    h = tid // 4
    p = tid % 4
    pltpu.sync_copy(esT.at[pl.ds(h * NP, NP)], colbuf)
    pltpu.sync_copy(edT.at[pl.ds(h * NP, NP)], acol)

    def _achunk(t, carry):
        off = (p * (NCH // 4) + t) * CH
        pltpu.sync_copy(srcidx.at[pl.ds(off, CH)], sbuf)
        pltpu.sync_copy(dstidx.at[pl.ds(off, CH)], dbuf)

        def _g(i, carry2):
            sl = pl.ds(i * 16, 16)
            es16 = plsc.load_gather(colbuf, [sbuf[sl]])
            ed16 = plsc.load_gather(acol, [dbuf[sl]])
            t0 = es16 + ed16
            exb[sl] = jnp.exp(jnp.maximum(t0, NEG * t0))
            return carry2
        lax.fori_loop(0, GP16, _g, 0)
        pltpu.sync_copy(exb, exout.at[pl.ds(h * E + off, CH)])
        return carry
    lax.fori_loop(0, NCH // 4, _achunk, 0)
    plsc.subcore_barrier()

    zero16 = jnp.broadcast_to(jnp.float32(0.0), (16,))
    for q in range(3):
        k = q * NS + s
        valid = k < 36
        kfeat = jnp.logical_and(valid, k < 32)

        @pl.when(kfeat)
        def _():
            pltpu.sync_copy(featT.at[pl.ds((c * 32 + k) * NP, NP)], colbuf)

        @pl.when(valid)
        def _():
            def _z(r, carry):
                acol[pl.ds(r * 16, 16)] = zero16
                return carry
            lax.fori_loop(0, NP // 16, _z, 0)

            hg = jnp.where(k < 32, k // 8, k - 32) + c * 4
            is_feat = k < 32

            def _bchunk(t, carry):
                off = t * CH
                pltpu.sync_copy(srcidx.at[pl.ds(off, CH)], sbuf)
                pltpu.sync_copy(dstidx.at[pl.ds(off, CH)], dbuf)
                pltpu.sync_copy(exout.at[pl.ds(hg * E + off, CH)], exb)

                def _g(i, carry2):
                    sl = pl.ds(i * 16, 16)
                    ex16 = exb[sl]
                    f16 = plsc.load_gather(colbuf, [sbuf[sl]])
                    v = jnp.where(is_feat, f16 * ex16, ex16)
                    plsc.addupdate_scatter(acol, [dbuf[sl]], v)
                    return carry2
                lax.fori_loop(0, GP16, _g, 0)
                return carry
            lax.fori_loop(0, NCH, _bchunk, 0)
            pltpu.sync_copy(acol, accout.at[pl.ds((c * 36 + k) * NP, NP)])


def _run_sc_edge(esT, edT, featT, src, dst):
    mesh = plsc.VectorSubcoreMesh(core_axis_name="c", subcore_axis_name="s")
    f = pl.kernel(
        _sc_edge_body,
        mesh=mesh,
        compiler_params=pltpu.CompilerParams(
            use_tc_tiling_on_sc=False, needs_layout_passes=False),
        out_type=[
            jax.ShapeDtypeStruct((H * E,), jnp.float32),
            jax.ShapeDtypeStruct((72 * NP,), jnp.float32),
        ],
        scratch_types=[
            pltpu.VMEM((NP,), jnp.float32),
            pltpu.VMEM((NP,), jnp.float32),
            pltpu.VMEM((CH,), jnp.int32),
            pltpu.VMEM((CH,), jnp.int32),
            pltpu.VMEM((CH,), jnp.float32),
        ],
    )
    return f(esT, edT, featT, src, dst)


def _final_body(a_ref, x_ref, resw_ref, resb_ref, rw_ref,
                relemb_ref, rpw_ref, rpb_ref, out_ref, relout_ref):
    a = a_ref[...]
    msg = jnp.concatenate([a[:, 0:32], a[:, 36:68]], axis=1)
    den = jnp.concatenate([a[:, 32:36], a[:, 68:72]], axis=1)
    den8 = jnp.broadcast_to(den.reshape(RB, H, 1), (RB, H, HD)).reshape(RB, 64)
    o = jnp.maximum(msg / (den8 + 1e-16), 0.0)
    alpha = 1.0 / (1.0 + jnp.exp(-rw_ref[0, 0]))
    res = jnp.dot(x_ref[...], resw_ref[...],
                  preferred_element_type=jnp.float32) + resb_ref[...]
    out_ref[...] = o * alpha + res * (1.0 - alpha)
    relout_ref[...] = jnp.dot(relemb_ref[...], rpw_ref[...],
                              preferred_element_type=jnp.float32) + rpb_ref[...]


def _run_final(accT, x, resw_t, resb2, rw2, relemb2, rpw_t, rpb2):
    return pl.pallas_call(
        _final_body,
        grid=(GRID,),
        in_specs=[
            pl.BlockSpec((RB, 72), lambda i: (i, 0)),
            pl.BlockSpec((RB, DIM), lambda i: (i, 0)),
            pl.BlockSpec((DIM, DIM), lambda i: (0, 0)),
            pl.BlockSpec((1, DIM), lambda i: (0, 0)),
            pl.BlockSpec((1, 1), lambda i: (0, 0)),
            pl.BlockSpec((1, DIM), lambda i: (0, 0)),
            pl.BlockSpec((DIM, DIM), lambda i: (0, 0)),
            pl.BlockSpec((1, DIM), lambda i: (0, 0)),
        ],
        out_specs=[
            pl.BlockSpec((RB, DIM), lambda i: (i, 0)),
            pl.BlockSpec((1, DIM), lambda i: (0, 0)),
        ],
        out_shape=[
            jax.ShapeDtypeStruct((N, DIM), jnp.float32),
            jax.ShapeDtypeStruct((1, DIM), jnp.float32),
        ],
    )(accT, x, resw_t, resb2, rw2, relemb2, rpw_t, rpb2)


def kernel(x, relation_embedding, W_src, W_dst, rel_trans_w, res_w, res_b,
           residual_weight, rel_prop_w, rel_prop_b, edge_index):
    f32 = jnp.float32
    rel_attn = (relation_embedding @ rel_trans_w).reshape(H, 2 * HD)
    attn_dst = rel_attn[:, :HD]
    attn_src = rel_attn[:, HD:]
    eye = jnp.eye(H, dtype=f32)
    Msrc = (attn_src[:, :, None] * eye[:, None, :]).reshape(H * HD, H)
    Mdst = (attn_dst[:, :, None] * eye[:, None, :]).reshape(H * HD, H)
    EsrcW = W_src @ Msrc
    EdstW = W_dst @ Mdst

    fs, es, ed = _run_proj(x, W_src, EsrcW, EdstW)
    pad = ((0, 0), (0, NP - N))
    featT = jnp.pad(fs.T, pad).reshape(DIM * NP)
    esT = jnp.pad(es.T, pad).reshape(H * NP)
    edT = jnp.pad(ed.T, pad).reshape(H * NP)

    _, accflat = _run_sc_edge(esT, edT, featT, edge_index[0], edge_index[1])
    accT = accflat.reshape(72, NP)[:, :N].T

    out, rel2d = _run_final(accT, x, res_w.T, res_b[None, :],
                            residual_weight.reshape(1, 1),
                            relation_embedding[None, :], rel_prop_w.T,
                            rel_prop_b[None, :])
    return out, rel2d.reshape(DIM)


# speedup vs baseline: 39.7590x; 1.5168x over previous
import jax
import jax.numpy as jnp
from jax import lax
from jax.experimental import pallas as pl
from jax.experimental.pallas import tpu as pltpu
from jax.experimental.pallas import tpu_sc as plsc

N = 50000
NP = 50048
E = 800000
DIM = 64
H = 8
HD = 8
NEG = 0.2
NS = 16
CH = 8000
NCH = E // CH
GP16 = CH // 16
RB = 1000
GRID = N // RB


def _proj_body(x_ref, wsrc_ref, esw_ref, edw_ref, fs_ref, es_ref, ed_ref):
    xb = x_ref[...]
    fs_ref[...] = jnp.dot(xb, wsrc_ref[...], preferred_element_type=jnp.float32)
    es_ref[...] = jnp.dot(xb, esw_ref[...], preferred_element_type=jnp.float32)
    ed_ref[...] = jnp.dot(xb, edw_ref[...], preferred_element_type=jnp.float32)


def _run_proj(x, W_src, EsrcW, EdstW):
    return pl.pallas_call(
        _proj_body,
        grid=(GRID,),
        in_specs=[
            pl.BlockSpec((RB, DIM), lambda i: (i, 0)),
            pl.BlockSpec((DIM, DIM), lambda i: (0, 0)),
            pl.BlockSpec((DIM, H), lambda i: (0, 0)),
            pl.BlockSpec((DIM, H), lambda i: (0, 0)),
        ],
        out_specs=[
            pl.BlockSpec((RB, DIM), lambda i: (i, 0)),
            pl.BlockSpec((RB, H), lambda i: (i, 0)),
            pl.BlockSpec((RB, H), lambda i: (i, 0)),
        ],
        out_shape=[
            jax.ShapeDtypeStruct((N, DIM), jnp.float32),
            jax.ShapeDtypeStruct((N, H), jnp.float32),
            jax.ShapeDtypeStruct((N, H), jnp.float32),
        ],
    )(x, W_src, EsrcW, EdstW)


def _sc_edge_body(esT, edT, featT, srcidx, dstidx,
                  exout, accout,
                  colbuf, acol, sbuf, dbuf, exb):
    c = lax.axis_index("c")
    s = lax.axis_index("s")
    tid = c * NS + s

    h = tid // 4
    p = tid % 4
    pltpu.sync_copy(esT.at[pl.ds(h * NP, NP)], colbuf)
    pltpu.sync_copy(edT.at[pl.ds(h * NP, NP)], acol)

    def _achunk(t, carry):
        off = (p * (NCH // 4) + t) * CH
        pltpu.sync_copy(srcidx.at[pl.ds(off, CH)], sbuf)
        pltpu.sync_copy(dstidx.at[pl.ds(off, CH)], dbuf)

        def _g(i, carry2):
            sl = pl.ds(i * 16, 16)
            es16 = plsc.load_gather(colbuf, [sbuf[sl]])
            ed16 = plsc.load_gather(acol, [dbuf[sl]])
            t0 = es16 + ed16
            exb[sl] = jnp.exp(jnp.maximum(t0, NEG * t0))
            return carry2
        lax.fori_loop(0, GP16, _g, 0)
        pltpu.sync_copy(exb, exout.at[pl.ds(h * E + off, CH)])
        return carry
    lax.fori_loop(0, NCH // 4, _achunk, 0)
    plsc.subcore_barrier()

    zero16 = jnp.broadcast_to(jnp.float32(0.0), (16,))
    for q in range(3):
        k = q * NS + s
        valid = k < 36
        kfeat = jnp.logical_and(valid, k < 32)

        @pl.when(kfeat)
        def _():
            pltpu.sync_copy(featT.at[pl.ds((c * 32 + k) * NP, NP)], colbuf)

        @pl.when(valid)
        def _():
            def _z(r, carry):
                acol[pl.ds(r * 16, 16)] = zero16
                return carry
            lax.fori_loop(0, NP // 16, _z, 0)

            hg = jnp.where(k < 32, k // 8, k - 32) + c * 4
            is_feat = k < 32

            def _bchunk(t, carry):
                off = t * CH
                pltpu.sync_copy(srcidx.at[pl.ds(off, CH)], sbuf)
                pltpu.sync_copy(dstidx.at[pl.ds(off, CH)], dbuf)
                pltpu.sync_copy(exout.at[pl.ds(hg * E + off, CH)], exb)

                def _g(i, carry2):
                    sl = pl.ds(i * 16, 16)
                    ex16 = exb[sl]
                    f16 = plsc.load_gather(colbuf, [sbuf[sl]])
                    v = jnp.where(is_feat, f16 * ex16, ex16)
                    plsc.addupdate_scatter(acol, [dbuf[sl]], v)
                    return carry2
                lax.fori_loop(0, GP16, _g, 0)
                return carry
            lax.fori_loop(0, NCH, _bchunk, 0)
            pltpu.sync_copy(acol, accout.at[pl.ds((c * 36 + k) * NP, NP)])


def _run_sc_edge(esT, edT, featT, src, dst):
    mesh = plsc.VectorSubcoreMesh(core_axis_name="c", subcore_axis_name="s")
    f = pl.kernel(
        _sc_edge_body,
        mesh=mesh,
        compiler_params=pltpu.CompilerParams(
            use_tc_tiling_on_sc=False, needs_layout_passes=False),
        out_type=[
            jax.ShapeDtypeStruct((H * E,), jnp.float32),
            jax.ShapeDtypeStruct((72 * NP,), jnp.float32),
        ],
        scratch_types=[
            pltpu.VMEM((NP,), jnp.float32),
            pltpu.VMEM((NP,), jnp.float32),
            pltpu.VMEM((CH,), jnp.int32),
            pltpu.VMEM((CH,), jnp.int32),
            pltpu.VMEM((CH,), jnp.float32),
        ],
    )
    return f(esT, edT, featT, src, dst)


def _final_body(a_ref, x_ref, resw_ref, resb_ref, rw_ref,
                relemb_ref, rpw_ref, rpb_ref, out_ref, relout_ref):
    a = a_ref[...]
    msg = jnp.concatenate([a[:, 0:32], a[:, 36:68]], axis=1)
    den = jnp.concatenate([a[:, 32:36], a[:, 68:72]], axis=1)
    den8 = jnp.broadcast_to(den.reshape(RB, H, 1), (RB, H, HD)).reshape(RB, 64)
    o = jnp.maximum(msg / (den8 + 1e-16), 0.0)
    alpha = 1.0 / (1.0 + jnp.exp(-rw_ref[0, 0]))
    res = jnp.dot(x_ref[...], resw_ref[...],
                  preferred_element_type=jnp.float32) + resb_ref[...]
    out_ref[...] = o * alpha + res * (1.0 - alpha)
    relout_ref[...] = jnp.dot(relemb_ref[...], rpw_ref[...],
                              preferred_element_type=jnp.float32) + rpb_ref[...]


def _run_final(accT, x, resw_t, resb2, rw2, relemb2, rpw_t, rpb2):
    return pl.pallas_call(
        _final_body,
        grid=(GRID,),
        in_specs=[
            pl.BlockSpec((RB, 72), lambda i: (i, 0)),
            pl.BlockSpec((RB, DIM), lambda i: (i, 0)),
            pl.BlockSpec((DIM, DIM), lambda i: (0, 0)),
            pl.BlockSpec((1, DIM), lambda i: (0, 0)),
            pl.BlockSpec((1, 1), lambda i: (0, 0)),
            pl.BlockSpec((1, DIM), lambda i: (0, 0)),
            pl.BlockSpec((DIM, DIM), lambda i: (0, 0)),
            pl.BlockSpec((1, DIM), lambda i: (0, 0)),
        ],
        out_specs=[
            pl.BlockSpec((RB, DIM), lambda i: (i, 0)),
            pl.BlockSpec((1, DIM), lambda i: (0, 0)),
        ],
        out_shape=[
            jax.ShapeDtypeStruct((N, DIM), jnp.float32),
            jax.ShapeDtypeStruct((1, DIM), jnp.float32),
        ],
    )(accT, x, resw_t, resb2, rw2, relemb2, rpw_t, rpb2)


def kernel(x, relation_embedding, W_src, W_dst, rel_trans_w, res_w, res_b,
           residual_weight, rel_prop_w, rel_prop_b, edge_index):
    f32 = jnp.float32
    rel_attn = (relation_embedding @ rel_trans_w).reshape(H, 2 * HD)
    attn_dst = rel_attn[:, :HD]
    attn_src = rel_attn[:, HD:]
    eye = jnp.eye(H, dtype=f32)
    Msrc = (attn_src[:, :, None] * eye[:, None, :]).reshape(H * HD, H)
    Mdst = (attn_dst[:, :, None] * eye[:, None, :]).reshape(H * HD, H)
    EsrcW = W_src @ Msrc
    EdstW = W_dst @ Mdst

    fs, es, ed = _run_proj(x, W_src, EsrcW, EdstW)
    pad = ((0, 0), (0, NP - N))
    featT = jnp.pad(fs.T, pad).reshape(DIM * NP)
    esT = jnp.pad(es.T, pad).reshape(H * NP)
    edT = jnp.pad(ed.T, pad).reshape(H * NP)

    _, accflat = _run_sc_edge(esT, edT, featT, edge_index[0], edge_index[1])
    accT = accflat.reshape(72, NP)[:, :N].T

    out, rel2d = _run_final(accT, x, res_w.T, res_b[None, :],
                            residual_weight.reshape(1, 1),
                            relation_embedding[None, :], rel_prop_w.T,
                            rel_prop_b[None, :])
    return out, rel2d.reshape(DIM)


# speedup vs baseline: 54.0403x; 1.3592x over previous
import jax
import jax.numpy as jnp
from jax import lax
from jax.experimental import pallas as pl
from jax.experimental.pallas import tpu as pltpu
from jax.experimental.pallas import tpu_sc as plsc

N = 50000
NP = 50048
E = 800000
DIM = 64
H = 8
HD = 8
NEG = 0.2
NS = 16
CH = 4000
NCH = E // CH
GP16 = CH // 16
RB = 1000
GRID = N // RB


def _proj_body(x_ref, wsrc_ref, esw_ref, edw_ref, fs_ref, es_ref, ed_ref):
    xb = x_ref[...]
    fs_ref[...] = jnp.dot(xb, wsrc_ref[...], preferred_element_type=jnp.float32)
    es_ref[...] = jnp.dot(xb, esw_ref[...], preferred_element_type=jnp.float32)
    ed_ref[...] = jnp.dot(xb, edw_ref[...], preferred_element_type=jnp.float32)


def _run_proj(x, W_src, EsrcW, EdstW):
    return pl.pallas_call(
        _proj_body,
        grid=(GRID,),
        in_specs=[
            pl.BlockSpec((RB, DIM), lambda i: (i, 0)),
            pl.BlockSpec((DIM, DIM), lambda i: (0, 0)),
            pl.BlockSpec((DIM, H), lambda i: (0, 0)),
            pl.BlockSpec((DIM, H), lambda i: (0, 0)),
        ],
        out_specs=[
            pl.BlockSpec((RB, DIM), lambda i: (i, 0)),
            pl.BlockSpec((RB, H), lambda i: (i, 0)),
            pl.BlockSpec((RB, H), lambda i: (i, 0)),
        ],
        out_shape=[
            jax.ShapeDtypeStruct((N, DIM), jnp.float32),
            jax.ShapeDtypeStruct((N, H), jnp.float32),
            jax.ShapeDtypeStruct((N, H), jnp.float32),
        ],
    )(x, W_src, EsrcW, EdstW)


def _sc_edge_body(esT, edT, featT, srcidx, dstidx,
                  exout, accout,
                  colbuf, acol, sbuf, dbuf, exb,
                  sbuf1, dbuf1, exb1, semA, semB):
    c = lax.axis_index("c")
    s = lax.axis_index("s")
    tid = c * NS + s

    h = tid // 4
    p = tid % 4
    pltpu.sync_copy(esT.at[pl.ds(h * NP, NP)], colbuf)
    pltpu.sync_copy(edT.at[pl.ds(h * NP, NP)], acol)

    def _achunk(t, carry):
        off = (p * (NCH // 4) + t) * CH
        pltpu.sync_copy(srcidx.at[pl.ds(off, CH)], sbuf)
        pltpu.sync_copy(dstidx.at[pl.ds(off, CH)], dbuf)

        def _g(i, carry2):
            sl = pl.ds(i * 16, 16)
            es16 = plsc.load_gather(colbuf, [sbuf[sl]])
            ed16 = plsc.load_gather(acol, [dbuf[sl]])
            t0 = es16 + ed16
            exb[sl] = jnp.exp(jnp.maximum(t0, NEG * t0))
            return carry2
        lax.fori_loop(0, GP16, _g, 0)
        pltpu.sync_copy(exb, exout.at[pl.ds(h * E + off, CH)])
        return carry
    lax.fori_loop(0, NCH // 4, _achunk, 0)
    plsc.subcore_barrier()

    zero16 = jnp.broadcast_to(jnp.float32(0.0), (16,))
    for q in range(3):
        k = q * NS + s
        valid = k < 36
        kfeat = jnp.logical_and(valid, k < 32)

        @pl.when(kfeat)
        def _():
            pltpu.sync_copy(featT.at[pl.ds((c * 32 + k) * NP, NP)], colbuf)

        @pl.when(valid)
        def _():
            def _z(r, carry):
                acol[pl.ds(r * 16, 16)] = zero16
                return carry
            lax.fori_loop(0, NP // 16, _z, 0)

            hg = jnp.where(k < 32, k // 8, k - 32) + c * 4
            is_feat = k < 32

            def _issue(off, sb, db, eb, sem):
                pltpu.async_copy(srcidx.at[pl.ds(off, CH)], sb, sem)
                pltpu.async_copy(dstidx.at[pl.ds(off, CH)], db, sem)
                pltpu.async_copy(exout.at[pl.ds(hg * E + off, CH)], eb, sem)

            def _wait(sb, db, eb, sem):
                pltpu.make_async_copy(srcidx.at[pl.ds(0, CH)], sb, sem).wait()
                pltpu.make_async_copy(dstidx.at[pl.ds(0, CH)], db, sem).wait()
                pltpu.make_async_copy(exout.at[pl.ds(0, CH)], eb, sem).wait()

            def _compute(sb, db, eb):
                def _g(i, carry2):
                    sl = pl.ds(i * 16, 16)
                    ex16 = eb[sl]
                    f16 = plsc.load_gather(colbuf, [sb[sl]])
                    v = jnp.where(is_feat, f16 * ex16, ex16)
                    plsc.addupdate_scatter(acol, [db[sl]], v)
                    return carry2
                lax.fori_loop(0, GP16, _g, 0)

            _issue(0, sbuf, dbuf, exb, semA)

            def _bchunk(t2, carry):
                t = t2 * 2
                _wait(sbuf, dbuf, exb, semA)
                _issue((t + 1) * CH, sbuf1, dbuf1, exb1, semB)
                _compute(sbuf, dbuf, exb)
                _wait(sbuf1, dbuf1, exb1, semB)

                @pl.when(t + 2 < NCH)
                def _():
                    _issue((t + 2) * CH, sbuf, dbuf, exb, semA)
                _compute(sbuf1, dbuf1, exb1)
                return carry
            lax.fori_loop(0, NCH // 2, _bchunk, 0)
            pltpu.sync_copy(acol, accout.at[pl.ds((c * 36 + k) * NP, NP)])


def _run_sc_edge(esT, edT, featT, src, dst):
    mesh = plsc.VectorSubcoreMesh(core_axis_name="c", subcore_axis_name="s")
    f = pl.kernel(
        _sc_edge_body,
        mesh=mesh,
        compiler_params=pltpu.CompilerParams(
            use_tc_tiling_on_sc=False, needs_layout_passes=False),
        out_type=[
            jax.ShapeDtypeStruct((H * E,), jnp.float32),
            jax.ShapeDtypeStruct((72 * NP,), jnp.float32),
        ],
        scratch_types=[
            pltpu.VMEM((NP,), jnp.float32),
            pltpu.VMEM((NP,), jnp.float32),
            pltpu.VMEM((CH,), jnp.int32),
            pltpu.VMEM((CH,), jnp.int32),
            pltpu.VMEM((CH,), jnp.float32),
            pltpu.VMEM((CH,), jnp.int32),
            pltpu.VMEM((CH,), jnp.int32),
            pltpu.VMEM((CH,), jnp.float32),
            pltpu.SemaphoreType.DMA,
            pltpu.SemaphoreType.DMA,
        ],
    )
    return f(esT, edT, featT, src, dst)


def _final_body(a_ref, x_ref, resw_ref, resb_ref, rw_ref,
                relemb_ref, rpw_ref, rpb_ref, out_ref, relout_ref):
    a = a_ref[...]
    msg = jnp.concatenate([a[:, 0:32], a[:, 36:68]], axis=1)
    den = jnp.concatenate([a[:, 32:36], a[:, 68:72]], axis=1)
    den8 = jnp.broadcast_to(den.reshape(RB, H, 1), (RB, H, HD)).reshape(RB, 64)
    o = jnp.maximum(msg / (den8 + 1e-16), 0.0)
    alpha = 1.0 / (1.0 + jnp.exp(-rw_ref[0, 0]))
    res = jnp.dot(x_ref[...], resw_ref[...],
                  preferred_element_type=jnp.float32) + resb_ref[...]
    out_ref[...] = o * alpha + res * (1.0 - alpha)
    relout_ref[...] = jnp.dot(relemb_ref[...], rpw_ref[...],
                              preferred_element_type=jnp.float32) + rpb_ref[...]


def _run_final(accT, x, resw_t, resb2, rw2, relemb2, rpw_t, rpb2):
    return pl.pallas_call(
        _final_body,
        grid=(GRID,),
        in_specs=[
            pl.BlockSpec((RB, 72), lambda i: (i, 0)),
            pl.BlockSpec((RB, DIM), lambda i: (i, 0)),
            pl.BlockSpec((DIM, DIM), lambda i: (0, 0)),
            pl.BlockSpec((1, DIM), lambda i: (0, 0)),
            pl.BlockSpec((1, 1), lambda i: (0, 0)),
            pl.BlockSpec((1, DIM), lambda i: (0, 0)),
            pl.BlockSpec((DIM, DIM), lambda i: (0, 0)),
            pl.BlockSpec((1, DIM), lambda i: (0, 0)),
        ],
        out_specs=[
            pl.BlockSpec((RB, DIM), lambda i: (i, 0)),
            pl.BlockSpec((1, DIM), lambda i: (0, 0)),
        ],
        out_shape=[
            jax.ShapeDtypeStruct((N, DIM), jnp.float32),
            jax.ShapeDtypeStruct((1, DIM), jnp.float32),
        ],
    )(accT, x, resw_t, resb2, rw2, relemb2, rpw_t, rpb2)


def kernel(x, relation_embedding, W_src, W_dst, rel_trans_w, res_w, res_b,
           residual_weight, rel_prop_w, rel_prop_b, edge_index):
    f32 = jnp.float32
    rel_attn = (relation_embedding @ rel_trans_w).reshape(H, 2 * HD)
    attn_dst = rel_attn[:, :HD]
    attn_src = rel_attn[:, HD:]
    eye = jnp.eye(H, dtype=f32)
    Msrc = (attn_src[:, :, None] * eye[:, None, :]).reshape(H * HD, H)
    Mdst = (attn_dst[:, :, None] * eye[:, None, :]).reshape(H * HD, H)
    EsrcW = W_src @ Msrc
    EdstW = W_dst @ Mdst

    fs, es, ed = _run_proj(x, W_src, EsrcW, EdstW)
    pad = ((0, 0), (0, NP - N))
    featT = jnp.pad(fs.T, pad).reshape(DIM * NP)
    esT = jnp.pad(es.T, pad).reshape(H * NP)
    edT = jnp.pad(ed.T, pad).reshape(H * NP)

    _, accflat = _run_sc_edge(esT, edT, featT, edge_index[0], edge_index[1])
    accT = accflat.reshape(72, NP)[:, :N].T

    out, rel2d = _run_final(accT, x, res_w.T, res_b[None, :],
                            residual_weight.reshape(1, 1),
                            relation_embedding[None, :], rel_prop_w.T,
                            rel_prop_b[None, :])
    return out, rel2d.reshape(DIM)


# speedup vs baseline: 58.2818x; 1.0785x over previous
import jax
import jax.numpy as jnp
from jax import lax
from jax.experimental import pallas as pl
from jax.experimental.pallas import tpu as pltpu
from jax.experimental.pallas import tpu_sc as plsc

N = 50000
NP = 50048
E = 800000
DIM = 64
H = 8
HD = 8
NEG = 0.2
NS = 16
CH = 4000
NCH = E // CH
GP16 = CH // 16
RB = 1000
GRID = N // RB


def _proj_body(x_ref, wsrc_ref, esw_ref, edw_ref, fs_ref, es_ref, ed_ref):
    xb = x_ref[...]
    fs_ref[...] = jnp.dot(xb, wsrc_ref[...], preferred_element_type=jnp.float32)
    es_ref[...] = jnp.dot(xb, esw_ref[...], preferred_element_type=jnp.float32)
    ed_ref[...] = jnp.dot(xb, edw_ref[...], preferred_element_type=jnp.float32)


def _run_proj(x, W_src, EsrcW, EdstW):
    return pl.pallas_call(
        _proj_body,
        grid=(GRID,),
        in_specs=[
            pl.BlockSpec((RB, DIM), lambda i: (i, 0)),
            pl.BlockSpec((DIM, DIM), lambda i: (0, 0)),
            pl.BlockSpec((DIM, H), lambda i: (0, 0)),
            pl.BlockSpec((DIM, H), lambda i: (0, 0)),
        ],
        out_specs=[
            pl.BlockSpec((RB, DIM), lambda i: (i, 0)),
            pl.BlockSpec((RB, H), lambda i: (i, 0)),
            pl.BlockSpec((RB, H), lambda i: (i, 0)),
        ],
        out_shape=[
            jax.ShapeDtypeStruct((N, DIM), jnp.float32),
            jax.ShapeDtypeStruct((N, H), jnp.float32),
            jax.ShapeDtypeStruct((N, H), jnp.float32),
        ],
    )(x, W_src, EsrcW, EdstW)


def _sc_edge_body(esT, edT, featT, srcidx, dstidx,
                  exout, accout,
                  colbuf, acol, sbuf, dbuf, exb,
                  sbuf1, dbuf1, exb1, semA, semB):
    c = lax.axis_index("c")
    s = lax.axis_index("s")
    tid = c * NS + s

    h = tid // 4
    p = tid % 4
    pltpu.sync_copy(esT.at[pl.ds(h * NP, NP)], colbuf)
    pltpu.sync_copy(edT.at[pl.ds(h * NP, NP)], acol)

    def _achunk(t, carry):
        off = (p * (NCH // 4) + t) * CH
        pltpu.sync_copy(srcidx.at[pl.ds(off, CH)], sbuf)
        pltpu.sync_copy(dstidx.at[pl.ds(off, CH)], dbuf)

        def _g(i, carry2):
            sl = pl.ds(i * 16, 16)
            es16 = plsc.load_gather(colbuf, [sbuf[sl]])
            ed16 = plsc.load_gather(acol, [dbuf[sl]])
            t0 = es16 + ed16
            exb[sl] = jnp.exp(jnp.maximum(t0, NEG * t0))
            return carry2
        lax.fori_loop(0, GP16, _g, 0)
        pltpu.sync_copy(exb, exout.at[pl.ds(h * E + off, CH)])
        return carry
    lax.fori_loop(0, NCH // 4, _achunk, 0)
    plsc.subcore_barrier()

    zero16 = jnp.broadcast_to(jnp.float32(0.0), (16,))

    def _zero_acol():
        def _z(r, carry):
            acol[pl.ds(r * 16, 16)] = zero16
            return carry
        lax.fori_loop(0, NP // 16, _z, 0)

    def _issue(off, hg, sb, db, eb, sem, with_src):
        if with_src:
            pltpu.async_copy(srcidx.at[pl.ds(off, CH)], sb, sem)
        pltpu.async_copy(dstidx.at[pl.ds(off, CH)], db, sem)
        pltpu.async_copy(exout.at[pl.ds(hg * E + off, CH)], eb, sem)

    def _wait(sb, db, eb, sem, with_src):
        if with_src:
            pltpu.make_async_copy(srcidx.at[pl.ds(0, CH)], sb, sem).wait()
        pltpu.make_async_copy(dstidx.at[pl.ds(0, CH)], db, sem).wait()
        pltpu.make_async_copy(exout.at[pl.ds(0, CH)], eb, sem).wait()

    def _compute(sb, db, eb, use_feat):
        def _g(i, carry2):
            sl = pl.ds(i * 16, 16)
            ex16 = eb[sl]
            if use_feat:
                v = plsc.load_gather(colbuf, [sb[sl]]) * ex16
            else:
                v = ex16
            plsc.addupdate_scatter(acol, [db[sl]], v)
            return carry2
        lax.fori_loop(0, GP16, _g, 0)

    def _sweep(c0, nch, hg, use_feat):
        _issue(c0 * CH, hg, sbuf, dbuf, exb, semA, use_feat)

        def _bchunk(t2, carry):
            t = c0 + t2 * 2
            _wait(sbuf, dbuf, exb, semA, use_feat)
            _issue((t + 1) * CH, hg, sbuf1, dbuf1, exb1, semB, use_feat)
            _compute(sbuf, dbuf, exb, use_feat)
            _wait(sbuf1, dbuf1, exb1, semB, use_feat)

            @pl.when(t + 2 < c0 + nch)
            def _():
                _issue((t + 2) * CH, hg, sbuf, dbuf, exb, semA, use_feat)
            _compute(sbuf1, dbuf1, exb1, use_feat)
            return carry
        lax.fori_loop(0, nch // 2, _bchunk, 0)

    for q in range(2):
        k = q * NS + s
        pltpu.sync_copy(featT.at[pl.ds((c * 32 + k) * NP, NP)], colbuf)
        _zero_acol()
        _sweep(0, NCH, k // 8 + c * 4, True)
        pltpu.sync_copy(acol, accout.at[pl.ds((c * 48 + k) * NP, NP)])

    _zero_acol()
    _sweep((s % 4) * (NCH // 4), NCH // 4, s // 4 + c * 4, False)
    pltpu.sync_copy(acol, accout.at[pl.ds((c * 48 + 32 + s) * NP, NP)])


def _run_sc_edge(esT, edT, featT, src, dst):
    mesh = plsc.VectorSubcoreMesh(core_axis_name="c", subcore_axis_name="s")
    f = pl.kernel(
        _sc_edge_body,
        mesh=mesh,
        compiler_params=pltpu.CompilerParams(
            use_tc_tiling_on_sc=False, needs_layout_passes=False),
        out_type=[
            jax.ShapeDtypeStruct((H * E,), jnp.float32),
            jax.ShapeDtypeStruct((96 * NP,), jnp.float32),
        ],
        scratch_types=[
            pltpu.VMEM((NP,), jnp.float32),
            pltpu.VMEM((NP,), jnp.float32),
            pltpu.VMEM((CH,), jnp.int32),
            pltpu.VMEM((CH,), jnp.int32),
            pltpu.VMEM((CH,), jnp.float32),
            pltpu.VMEM((CH,), jnp.int32),
            pltpu.VMEM((CH,), jnp.int32),
            pltpu.VMEM((CH,), jnp.float32),
            pltpu.SemaphoreType.DMA,
            pltpu.SemaphoreType.DMA,
        ],
    )
    return f(esT, edT, featT, src, dst)


def _final_body(a_ref, x_ref, resw_ref, resb_ref, rw_ref,
                relemb_ref, rpw_ref, rpb_ref, out_ref, relout_ref):
    a = a_ref[...]
    msg = jnp.concatenate([a[:, 0:32], a[:, 48:80]], axis=1)
    den = jnp.concatenate([a[:, 32:48], a[:, 80:96]],
                          axis=1).reshape(RB, H, 4).sum(-1)
    den8 = jnp.broadcast_to(den.reshape(RB, H, 1), (RB, H, HD)).reshape(RB, 64)
    o = jnp.maximum(msg / (den8 + 1e-16), 0.0)
    alpha = 1.0 / (1.0 + jnp.exp(-rw_ref[0, 0]))
    res = jnp.dot(x_ref[...], resw_ref[...],
                  preferred_element_type=jnp.float32) + resb_ref[...]
    out_ref[...] = o * alpha + res * (1.0 - alpha)
    relout_ref[...] = jnp.dot(relemb_ref[...], rpw_ref[...],
                              preferred_element_type=jnp.float32) + rpb_ref[...]


def _run_final(accT, x, resw_t, resb2, rw2, relemb2, rpw_t, rpb2):
    return pl.pallas_call(
        _final_body,
        grid=(GRID,),
        in_specs=[
            pl.BlockSpec((RB, 96), lambda i: (i, 0)),
            pl.BlockSpec((RB, DIM), lambda i: (i, 0)),
            pl.BlockSpec((DIM, DIM), lambda i: (0, 0)),
            pl.BlockSpec((1, DIM), lambda i: (0, 0)),
            pl.BlockSpec((1, 1), lambda i: (0, 0)),
            pl.BlockSpec((1, DIM), lambda i: (0, 0)),
            pl.BlockSpec((DIM, DIM), lambda i: (0, 0)),
            pl.BlockSpec((1, DIM), lambda i: (0, 0)),
        ],
        out_specs=[
            pl.BlockSpec((RB, DIM), lambda i: (i, 0)),
            pl.BlockSpec((1, DIM), lambda i: (0, 0)),
        ],
        out_shape=[
            jax.ShapeDtypeStruct((N, DIM), jnp.float32),
            jax.ShapeDtypeStruct((1, DIM), jnp.float32),
        ],
    )(accT, x, resw_t, resb2, rw2, relemb2, rpw_t, rpb2)


def kernel(x, relation_embedding, W_src, W_dst, rel_trans_w, res_w, res_b,
           residual_weight, rel_prop_w, rel_prop_b, edge_index):
    f32 = jnp.float32
    rel_attn = (relation_embedding @ rel_trans_w).reshape(H, 2 * HD)
    attn_dst = rel_attn[:, :HD]
    attn_src = rel_attn[:, HD:]
    eye = jnp.eye(H, dtype=f32)
    Msrc = (attn_src[:, :, None] * eye[:, None, :]).reshape(H * HD, H)
    Mdst = (attn_dst[:, :, None] * eye[:, None, :]).reshape(H * HD, H)
    EsrcW = W_src @ Msrc
    EdstW = W_dst @ Mdst

    fs, es, ed = _run_proj(x, W_src, EsrcW, EdstW)
    pad = ((0, 0), (0, NP - N))
    featT = jnp.pad(fs.T, pad).reshape(DIM * NP)
    esT = jnp.pad(es.T, pad).reshape(H * NP)
    edT = jnp.pad(ed.T, pad).reshape(H * NP)

    _, accflat = _run_sc_edge(esT, edT, featT, edge_index[0], edge_index[1])
    accT = accflat.reshape(96, NP)[:, :N].T

    out, rel2d = _run_final(accT, x, res_w.T, res_b[None, :],
                            residual_weight.reshape(1, 1),
                            relation_embedding[None, :], rel_prop_w.T,
                            rel_prop_b[None, :])
    return out, rel2d.reshape(DIM)


# speedup vs baseline: 59.5206x; 1.0213x over previous
import jax
import jax.numpy as jnp
from jax import lax
from jax.experimental import pallas as pl
from jax.experimental.pallas import tpu as pltpu
from jax.experimental.pallas import tpu_sc as plsc

N = 50000
NP = 50048
E = 800000
DIM = 64
H = 8
HD = 8
NEG = 0.2
NS = 16
CH = 4000
NCH = E // CH
GP16 = CH // 16
RB = 1000
GRID = N // RB


def _proj_body(x_ref, wsrc_ref, esw_ref, edw_ref, fs_ref, es_ref, ed_ref):
    xb = x_ref[...]
    fs_ref[...] = jnp.dot(xb, wsrc_ref[...], preferred_element_type=jnp.float32)
    es_ref[...] = jnp.dot(xb, esw_ref[...], preferred_element_type=jnp.float32)
    ed_ref[...] = jnp.dot(xb, edw_ref[...], preferred_element_type=jnp.float32)


def _run_proj(x, W_src, EsrcW, EdstW):
    return pl.pallas_call(
        _proj_body,
        grid=(GRID,),
        in_specs=[
            pl.BlockSpec((RB, DIM), lambda i: (i, 0)),
            pl.BlockSpec((DIM, DIM), lambda i: (0, 0)),
            pl.BlockSpec((DIM, H), lambda i: (0, 0)),
            pl.BlockSpec((DIM, H), lambda i: (0, 0)),
        ],
        out_specs=[
            pl.BlockSpec((RB, DIM), lambda i: (i, 0)),
            pl.BlockSpec((RB, H), lambda i: (i, 0)),
            pl.BlockSpec((RB, H), lambda i: (i, 0)),
        ],
        out_shape=[
            jax.ShapeDtypeStruct((N, DIM), jnp.float32),
            jax.ShapeDtypeStruct((N, H), jnp.float32),
            jax.ShapeDtypeStruct((N, H), jnp.float32),
        ],
    )(x, W_src, EsrcW, EdstW)


def _sc_edge_body(esT, edT, featT, srcidx, dstidx,
                  exout, accout,
                  colbuf, acol, sbuf, dbuf, exb,
                  sbuf1, dbuf1, exb1, semA, semB):
    c = lax.axis_index("c")
    s = lax.axis_index("s")
    tid = c * NS + s

    h = tid // 4
    p = tid % 4
    pltpu.sync_copy(esT.at[pl.ds(h * NP, NP)], colbuf)
    pltpu.sync_copy(edT.at[pl.ds(h * NP, NP)], acol)

    def _achunk(t, carry):
        off = (p * (NCH // 4) + t) * CH
        pltpu.sync_copy(srcidx.at[pl.ds(off, CH)], sbuf)
        pltpu.sync_copy(dstidx.at[pl.ds(off, CH)], dbuf)

        def _g(i, carry2):
            for u in range(5):
                sl = pl.ds((i * 5 + u) * 16, 16)
                es16 = plsc.load_gather(colbuf, [sbuf[sl]])
                ed16 = plsc.load_gather(acol, [dbuf[sl]])
                t0 = es16 + ed16
                exb[sl] = jnp.exp(jnp.maximum(t0, NEG * t0))
            return carry2
        lax.fori_loop(0, GP16 // 5, _g, 0)
        pltpu.sync_copy(exb, exout.at[pl.ds(h * E + off, CH)])
        return carry
    lax.fori_loop(0, NCH // 4, _achunk, 0)
    plsc.subcore_barrier()

    zero16 = jnp.broadcast_to(jnp.float32(0.0), (16,))

    def _zero_acol():
        def _z(r, carry):
            acol[pl.ds(r * 16, 16)] = zero16
            return carry
        lax.fori_loop(0, NP // 16, _z, 0)

    def _issue(off, hg, sb, db, eb, sem, with_src):
        if with_src:
            pltpu.async_copy(srcidx.at[pl.ds(off, CH)], sb, sem)
        pltpu.async_copy(dstidx.at[pl.ds(off, CH)], db, sem)
        pltpu.async_copy(exout.at[pl.ds(hg * E + off, CH)], eb, sem)

    def _wait(sb, db, eb, sem, with_src):
        if with_src:
            pltpu.make_async_copy(srcidx.at[pl.ds(0, CH)], sb, sem).wait()
        pltpu.make_async_copy(dstidx.at[pl.ds(0, CH)], db, sem).wait()
        pltpu.make_async_copy(exout.at[pl.ds(0, CH)], eb, sem).wait()

    def _compute(sb, db, eb, use_feat):
        def _g(i, carry2):
            for u in range(5):
                sl = pl.ds((i * 5 + u) * 16, 16)
                ex16 = eb[sl]
                if use_feat:
                    v = plsc.load_gather(colbuf, [sb[sl]]) * ex16
                else:
                    v = ex16
                plsc.addupdate_scatter(acol, [db[sl]], v)
            return carry2
        lax.fori_loop(0, GP16 // 5, _g, 0)

    def _sweep(c0, nch, hg, use_feat):
        _issue(c0 * CH, hg, sbuf, dbuf, exb, semA, use_feat)

        def _bchunk(t2, carry):
            t = c0 + t2 * 2
            _wait(sbuf, dbuf, exb, semA, use_feat)
            _issue((t + 1) * CH, hg, sbuf1, dbuf1, exb1, semB, use_feat)
            _compute(sbuf, dbuf, exb, use_feat)
            _wait(sbuf1, dbuf1, exb1, semB, use_feat)

            @pl.when(t + 2 < c0 + nch)
            def _():
                _issue((t + 2) * CH, hg, sbuf, dbuf, exb, semA, use_feat)
            _compute(sbuf1, dbuf1, exb1, use_feat)
            return carry
        lax.fori_loop(0, nch // 2, _bchunk, 0)

    for q in range(2):
        k = q * NS + s
        pltpu.sync_copy(featT.at[pl.ds((c * 32 + k) * NP, NP)], colbuf)
        _zero_acol()
        _sweep(0, NCH, k // 8 + c * 4, True)
        pltpu.sync_copy(acol, accout.at[pl.ds((c * 48 + k) * NP, NP)])

    _zero_acol()
    _sweep((s % 4) * (NCH // 4), NCH // 4, s // 4 + c * 4, False)
    pltpu.sync_copy(acol, accout.at[pl.ds((c * 48 + 32 + s) * NP, NP)])


def _run_sc_edge(esT, edT, featT, src, dst):
    mesh = plsc.VectorSubcoreMesh(core_axis_name="c", subcore_axis_name="s")
    f = pl.kernel(
        _sc_edge_body,
        mesh=mesh,
        compiler_params=pltpu.CompilerParams(
            use_tc_tiling_on_sc=False, needs_layout_passes=False),
        out_type=[
            jax.ShapeDtypeStruct((H * E,), jnp.float32),
            jax.ShapeDtypeStruct((96 * NP,), jnp.float32),
        ],
        scratch_types=[
            pltpu.VMEM((NP,), jnp.float32),
            pltpu.VMEM((NP,), jnp.float32),
            pltpu.VMEM((CH,), jnp.int32),
            pltpu.VMEM((CH,), jnp.int32),
            pltpu.VMEM((CH,), jnp.float32),
            pltpu.VMEM((CH,), jnp.int32),
            pltpu.VMEM((CH,), jnp.int32),
            pltpu.VMEM((CH,), jnp.float32),
            pltpu.SemaphoreType.DMA,
            pltpu.SemaphoreType.DMA,
        ],
    )
    return f(esT, edT, featT, src, dst)


def _final_body(a_ref, x_ref, resw_ref, resb_ref, rw_ref,
                relemb_ref, rpw_ref, rpb_ref, out_ref, relout_ref):
    a = a_ref[...]
    msg = jnp.concatenate([a[:, 0:32], a[:, 48:80]], axis=1)
    den = jnp.concatenate([a[:, 32:48], a[:, 80:96]],
                          axis=1).reshape(RB, H, 4).sum(-1)
    den8 = jnp.broadcast_to(den.reshape(RB, H, 1), (RB, H, HD)).reshape(RB, 64)
    o = jnp.maximum(msg / (den8 + 1e-16), 0.0)
    alpha = 1.0 / (1.0 + jnp.exp(-rw_ref[0, 0]))
    res = jnp.dot(x_ref[...], resw_ref[...],
                  preferred_element_type=jnp.float32) + resb_ref[...]
    out_ref[...] = o * alpha + res * (1.0 - alpha)
    relout_ref[...] = jnp.dot(relemb_ref[...], rpw_ref[...],
                              preferred_element_type=jnp.float32) + rpb_ref[...]


def _run_final(accT, x, resw_t, resb2, rw2, relemb2, rpw_t, rpb2):
    return pl.pallas_call(
        _final_body,
        grid=(GRID,),
        in_specs=[
            pl.BlockSpec((RB, 96), lambda i: (i, 0)),
            pl.BlockSpec((RB, DIM), lambda i: (i, 0)),
            pl.BlockSpec((DIM, DIM), lambda i: (0, 0)),
            pl.BlockSpec((1, DIM), lambda i: (0, 0)),
            pl.BlockSpec((1, 1), lambda i: (0, 0)),
            pl.BlockSpec((1, DIM), lambda i: (0, 0)),
            pl.BlockSpec((DIM, DIM), lambda i: (0, 0)),
            pl.BlockSpec((1, DIM), lambda i: (0, 0)),
        ],
        out_specs=[
            pl.BlockSpec((RB, DIM), lambda i: (i, 0)),
            pl.BlockSpec((1, DIM), lambda i: (0, 0)),
        ],
        out_shape=[
            jax.ShapeDtypeStruct((N, DIM), jnp.float32),
            jax.ShapeDtypeStruct((1, DIM), jnp.float32),
        ],
    )(accT, x, resw_t, resb2, rw2, relemb2, rpw_t, rpb2)


def kernel(x, relation_embedding, W_src, W_dst, rel_trans_w, res_w, res_b,
           residual_weight, rel_prop_w, rel_prop_b, edge_index):
    f32 = jnp.float32
    rel_attn = (relation_embedding @ rel_trans_w).reshape(H, 2 * HD)
    attn_dst = rel_attn[:, :HD]
    attn_src = rel_attn[:, HD:]
    eye = jnp.eye(H, dtype=f32)
    Msrc = (attn_src[:, :, None] * eye[:, None, :]).reshape(H * HD, H)
    Mdst = (attn_dst[:, :, None] * eye[:, None, :]).reshape(H * HD, H)
    EsrcW = W_src @ Msrc
    EdstW = W_dst @ Mdst

    fs, es, ed = _run_proj(x, W_src, EsrcW, EdstW)
    pad = ((0, 0), (0, NP - N))
    featT = jnp.pad(fs.T, pad).reshape(DIM * NP)
    esT = jnp.pad(es.T, pad).reshape(H * NP)
    edT = jnp.pad(ed.T, pad).reshape(H * NP)

    _, accflat = _run_sc_edge(esT, edT, featT, edge_index[0], edge_index[1])
    accT = accflat.reshape(96, NP)[:, :N].T

    out, rel2d = _run_final(accT, x, res_w.T, res_b[None, :],
                            residual_weight.reshape(1, 1),
                            relation_embedding[None, :], rel_prop_w.T,
                            rel_prop_b[None, :])
    return out, rel2d.reshape(DIM)
